# Initial kernel scaffold; baseline (speedup 1.0000x reference)
import jax
import jax.numpy as jnp
from jax.experimental import pallas as pl


def kernel(item_type, item, emb_table, W, b):
    raise NotImplementedError("write your pallas kernel here")


# speedup vs baseline: 2.0027x; 2.0027x over previous
import jax
import jax.numpy as jnp
from jax.experimental import pallas as pl
from jax.experimental.pallas import tpu as pltpu

_NI = 12


def _fused_block(t_ref, x_ref, emb_ref, w_ref, b_ref, o_ref):
    n_type = emb_ref.shape[0]
    n_emb = emb_ref.shape[1]
    t = t_ref[...]
    oh = (t == jax.lax.broadcasted_iota(jnp.int32, (1, n_type), 1)).astype(
        jnp.float32
    )
    x = x_ref[...]
    emb_w = jnp.dot(
        emb_ref[...], w_ref[0:n_emb, :], preferred_element_type=jnp.float32
    )
    inp = jnp.concatenate([oh, x], axis=1)
    wc = jnp.concatenate([emb_w, w_ref[n_emb:, :]], axis=0)
    h = jnp.dot(inp, wc, preferred_element_type=jnp.float32) + b_ref[...]
    h = jnp.maximum(h, 0.0)
    o_ref[...] = jnp.max(h.reshape(-1, _NI, h.shape[-1]), axis=1)


def kernel(item_type, item, emb_table, W, b):
    bs, na, ni = item_type.shape
    nf = item.shape[-1]
    hidden = W.shape[-1]
    rows = bs * na * ni

    t2 = item_type.reshape(rows, 1).astype(jnp.int32)
    x2 = item.reshape(rows, nf)
    b2 = b.reshape(1, hidden)

    groups_per_block = 1024
    r_block = groups_per_block * ni
    grid = rows // r_block

    out = pl.pallas_call(
        _fused_block,
        grid=(grid,),
        in_specs=[
            pl.BlockSpec((r_block, 1), lambda i: (i, 0)),
            pl.BlockSpec((r_block, nf), lambda i: (i, 0)),
            pl.BlockSpec(emb_table.shape, lambda i: (0, 0)),
            pl.BlockSpec(W.shape, lambda i: (0, 0)),
            pl.BlockSpec((1, hidden), lambda i: (0, 0)),
        ],
        out_specs=pl.BlockSpec((groups_per_block, hidden), lambda i: (i, 0)),
        out_shape=jax.ShapeDtypeStruct((bs * na, hidden), jnp.float32),
        compiler_params=pltpu.CompilerParams(
            dimension_semantics=("parallel",),
        ),
    )(t2, x2, emb_table, W, b2)

    return out.reshape(bs, na, hidden)


# speedup vs baseline: 5.1096x; 2.5513x over previous
import jax
import jax.numpy as jnp
from jax.experimental import pallas as pl
from jax.experimental.pallas import tpu as pltpu


def _fused_block(t_ref, x_ref, emb_ref, w_ref, b_ref, o_ref):
    blk, na, ni, nf = x_ref.shape
    n_type, n_emb = emb_ref.shape
    hidden = w_ref.shape[1]
    rows = blk * na

    emb_w = jnp.dot(
        emb_ref[...], w_ref[0:n_emb, :], preferred_element_type=jnp.float32
    )
    wc = jnp.concatenate([emb_w, w_ref[n_emb:, :]], axis=0)

    t4 = t_ref[...]
    x4 = x_ref[...]
    iota_t = jax.lax.broadcasted_iota(jnp.int32, (1, n_type), 1)

    acc = None
    for i in range(ni):
        ti = t4[:, :, i].reshape(rows, 1)
        xi = x4[:, :, i, :].reshape(rows, nf)
        oh = (ti == iota_t).astype(jnp.float32)
        inp = jnp.concatenate([oh, xi], axis=1)
        hi = jnp.dot(inp, wc, preferred_element_type=jnp.float32)
        acc = hi if acc is None else jnp.maximum(acc, hi)

    out = jnp.maximum(acc + b_ref[...], 0.0)
    o_ref[...] = out.reshape(blk, na, hidden)


def kernel(item_type, item, emb_table, W, b):
    bs, na, ni = item_type.shape
    nf = item.shape[-1]
    hidden = W.shape[-1]

    blk = 64
    grid = bs // blk

    out = pl.pallas_call(
        _fused_block,
        grid=(grid,),
        in_specs=[
            pl.BlockSpec((blk, na, ni), lambda i: (i, 0, 0)),
            pl.BlockSpec((blk, na, ni, nf), lambda i: (i, 0, 0, 0)),
            pl.BlockSpec(emb_table.shape, lambda i: (0, 0)),
            pl.BlockSpec(W.shape, lambda i: (0, 0)),
            pl.BlockSpec((1, hidden), lambda i: (0, 0)),
        ],
        out_specs=pl.BlockSpec((blk, na, hidden), lambda i: (i, 0, 0)),
        out_shape=jax.ShapeDtypeStruct((bs, na, hidden), jnp.float32),
        compiler_params=pltpu.CompilerParams(
            dimension_semantics=("parallel",),
        ),
    )(item_type.astype(jnp.int32), item, emb_table, W, b.reshape(1, hidden))

    return out


# speedup vs baseline: 18.4235x; 3.6057x over previous
import jax
import jax.numpy as jnp
from jax.experimental import pallas as pl
from jax.experimental.pallas import tpu as pltpu


def _fused_block(t_ref, x_ref, emb_ref, w_ref, b_ref, o_ref):
    ni, nf, lanes = x_ref.shape
    n_type, n_emb = emb_ref.shape
    hidden = w_ref.shape[1]

    emb_w = jnp.dot(
        emb_ref[...], w_ref[0:n_emb, :], preferred_element_type=jnp.float32
    )
    wc = jnp.concatenate([w_ref[n_emb:, :], emb_w], axis=0)
    wc_t = wc.T

    iota_t = jax.lax.broadcasted_iota(jnp.int32, (n_type, 1), 0)

    acc = None
    for i in range(ni):
        xi = x_ref[i]
        ti = t_ref[i].reshape(1, lanes)
        oh = (ti == iota_t).astype(jnp.float32)
        rhs = jnp.concatenate([xi, oh], axis=0)
        hi = jnp.dot(wc_t, rhs, preferred_element_type=jnp.float32)
        acc = hi if acc is None else jnp.maximum(acc, hi)

    o_ref[...] = jnp.maximum(acc + b_ref[...], 0.0)


def kernel(item_type, item, emb_table, W, b):
    bs, na, ni = item_type.shape
    nf = item.shape[-1]
    hidden = W.shape[-1]
    cols = na * bs

    x_t = jnp.transpose(item, (2, 3, 1, 0)).reshape(ni, nf, cols)
    t_t = jnp.transpose(item_type.astype(jnp.int32), (2, 1, 0)).reshape(ni, cols)

    lane_blk = 1024
    grid = cols // lane_blk

    out = pl.pallas_call(
        _fused_block,
        grid=(grid,),
        in_specs=[
            pl.BlockSpec((ni, lane_blk), lambda j: (0, j)),
            pl.BlockSpec((ni, nf, lane_blk), lambda j: (0, 0, j)),
            pl.BlockSpec(emb_table.shape, lambda j: (0, 0)),
            pl.BlockSpec(W.shape, lambda j: (0, 0)),
            pl.BlockSpec((hidden, 1), lambda j: (0, 0)),
        ],
        out_specs=pl.BlockSpec((hidden, lane_blk), lambda j: (0, j)),
        out_shape=jax.ShapeDtypeStruct((hidden, cols), jnp.float32),
        compiler_params=pltpu.CompilerParams(
            dimension_semantics=("parallel",),
        ),
    )(t_t, x_t, emb_table, W, b.reshape(hidden, 1))

    return out.reshape(hidden, na, bs).transpose(2, 1, 0)


# speedup vs baseline: 21.4481x; 1.1642x over previous
import jax
import jax.numpy as jnp
from jax.experimental import pallas as pl
from jax.experimental.pallas import tpu as pltpu


def _fused_block(t_ref, x_ref, emb_ref, w_ref, b_ref, o_ref):
    ni, nf, lanes = x_ref.shape
    n_type, n_emb = emb_ref.shape
    hidden = w_ref.shape[1]
    a_idx = pl.program_id(1)

    emb_w = jnp.dot(
        emb_ref[...], w_ref[0:n_emb, :], preferred_element_type=jnp.float32
    )
    wc = jnp.concatenate([w_ref[n_emb:, :], emb_w], axis=0)
    wc_t = wc.T

    iota_t = jax.lax.broadcasted_iota(jnp.int32, (n_type, 1), 0)
    ta = t_ref[:, pl.ds(a_idx, 1), :]

    acc = None
    for i in range(ni):
        xi = x_ref[i]
        ti = ta[i]
        oh = (ti == iota_t).astype(jnp.float32)
        rhs = jnp.concatenate([xi, oh], axis=0)
        hi = jnp.dot(wc_t, rhs, preferred_element_type=jnp.float32)
        acc = hi if acc is None else jnp.maximum(acc, hi)

    out = jnp.maximum(acc + b_ref[...], 0.0)
    o_ref[...] = out.reshape(1, hidden, lanes)


def kernel(item_type, item, emb_table, W, b):
    bs, na, ni = item_type.shape
    nf = item.shape[-1]
    hidden = W.shape[-1]

    x_t = jnp.transpose(item, (2, 3, 1, 0)).reshape(ni, nf, na * bs)
    t_t = jnp.transpose(item_type.astype(jnp.int32), (2, 1, 0))

    lane_blk = 1024
    jgrid = bs // lane_blk

    out = pl.pallas_call(
        _fused_block,
        grid=(jgrid, na),
        in_specs=[
            pl.BlockSpec((ni, na, lane_blk), lambda j, a: (0, 0, j)),
            pl.BlockSpec(
                (ni, nf, lane_blk), lambda j, a, _g=jgrid: (0, 0, a * _g + j)
            ),
            pl.BlockSpec(emb_table.shape, lambda j, a: (0, 0)),
            pl.BlockSpec(W.shape, lambda j, a: (0, 0)),
            pl.BlockSpec((hidden, 1), lambda j, a: (0, 0)),
        ],
        out_specs=pl.BlockSpec((1, hidden, lane_blk), lambda j, a: (a, 0, j)),
        out_shape=jax.ShapeDtypeStruct((na, hidden, bs), jnp.float32),
        compiler_params=pltpu.CompilerParams(
            dimension_semantics=("parallel", "parallel"),
        ),
    )(t_t, x_t, emb_table, W, b.reshape(hidden, 1))

    return out.transpose(2, 0, 1)


# speedup vs baseline: 24.0075x; 1.1193x over previous
import jax
import jax.numpy as jnp
from jax.experimental import pallas as pl
from jax.experimental.pallas import tpu as pltpu


def _fused_block(t_ref, x_ref, emb_ref, w_ref, b_ref, o_ref):
    ni, nf, lanes = x_ref.shape
    n_type, n_emb = emb_ref.shape
    hidden = w_ref.shape[1]
    a_idx = pl.program_id(1)

    emb_w = jnp.dot(
        emb_ref[...], w_ref[0:n_emb, :], preferred_element_type=jnp.float32
    )
    wc = jnp.concatenate([w_ref[n_emb:, :], emb_w], axis=0)
    wc_t = wc.T

    iota_t = jax.lax.broadcasted_iota(jnp.int32, (n_type, 1), 0)

    lt = 256
    for c in range(lanes // lt):
        sl = slice(c * lt, (c + 1) * lt)
        ta = t_ref[:, pl.ds(a_idx, 1), sl]
        acc = None
        for i in range(ni):
            xi = x_ref[i, :, sl]
            ti = ta[i]
            oh = (ti == iota_t).astype(jnp.float32)
            rhs = jnp.concatenate([xi, oh], axis=0)
            hi = jnp.dot(wc_t, rhs, preferred_element_type=jnp.float32)
            acc = hi if acc is None else jnp.maximum(acc, hi)
        out = jnp.maximum(acc + b_ref[...], 0.0)
        o_ref[0, :, sl] = out


def kernel(item_type, item, emb_table, W, b):
    bs, na, ni = item_type.shape
    nf = item.shape[-1]
    hidden = W.shape[-1]

    x_t = jnp.transpose(item, (2, 3, 1, 0)).reshape(ni, nf, na * bs)
    t_t = jnp.transpose(item_type.astype(jnp.int32), (2, 1, 0))

    lane_blk = 2048
    jgrid = bs // lane_blk

    out = pl.pallas_call(
        _fused_block,
        grid=(jgrid, na),
        in_specs=[
            pl.BlockSpec((ni, na, lane_blk), lambda j, a: (0, 0, j)),
            pl.BlockSpec(
                (ni, nf, lane_blk), lambda j, a, _g=jgrid: (0, 0, a * _g + j)
            ),
            pl.BlockSpec(emb_table.shape, lambda j, a: (0, 0)),
            pl.BlockSpec(W.shape, lambda j, a: (0, 0)),
            pl.BlockSpec((hidden, 1), lambda j, a: (0, 0)),
        ],
        out_specs=pl.BlockSpec((1, hidden, lane_blk), lambda j, a: (a, 0, j)),
        out_shape=jax.ShapeDtypeStruct((na, hidden, bs), jnp.float32),
        compiler_params=pltpu.CompilerParams(
            dimension_semantics=("parallel", "parallel"),
        ),
    )(t_t, x_t, emb_table, W, b.reshape(hidden, 1))

    return out.transpose(2, 0, 1)


# speedup vs baseline: 25.0786x; 1.0446x over previous
import jax
import jax.numpy as jnp
from jax.experimental import pallas as pl
from jax.experimental.pallas import tpu as pltpu


def _fused_block(t_ref, x_ref, emb_ref, w_ref, b_ref, o_ref):
    ni = t_ref.shape[0]
    nf = x_ref.shape[0] // ni
    lanes = x_ref.shape[1]
    n_type, n_emb = emb_ref.shape
    hidden = w_ref.shape[1]
    a_idx = pl.program_id(1)

    emb_w = jnp.dot(
        emb_ref[...], w_ref[0:n_emb, :], preferred_element_type=jnp.float32
    )
    wc = jnp.concatenate([w_ref[n_emb:, :], emb_w], axis=0)
    wc_t = wc.T

    iota_t = jax.lax.broadcasted_iota(jnp.int32, (n_type, 1), 0)

    lt = 256
    for c in range(lanes // lt):
        sl = slice(c * lt, (c + 1) * lt)
        ta = t_ref[:, pl.ds(a_idx, 1), sl]
        acc = None
        for i in range(ni):
            xi = x_ref[i * nf:(i + 1) * nf, sl]
            ti = ta[i]
            oh = (ti == iota_t).astype(jnp.float32)
            rhs = jnp.concatenate([xi, oh], axis=0)
            hi = jnp.dot(wc_t, rhs, preferred_element_type=jnp.float32)
            acc = hi if acc is None else jnp.maximum(acc, hi)
        out = jnp.maximum(acc + b_ref[...], 0.0)
        o_ref[0, :, sl] = out


def kernel(item_type, item, emb_table, W, b):
    bs, na, ni = item_type.shape
    nf = item.shape[-1]
    hidden = W.shape[-1]

    x_t = jnp.transpose(item, (2, 3, 1, 0)).reshape(ni * nf, na * bs)
    t_t = jnp.transpose(item_type.astype(jnp.int32), (2, 1, 0))

    lane_blk = 2048
    jgrid = bs // lane_blk

    out = pl.pallas_call(
        _fused_block,
        grid=(jgrid, na),
        in_specs=[
            pl.BlockSpec((ni, na, lane_blk), lambda j, a: (0, 0, j)),
            pl.BlockSpec(
                (ni * nf, lane_blk), lambda j, a, _g=jgrid: (0, a * _g + j)
            ),
            pl.BlockSpec(emb_table.shape, lambda j, a: (0, 0)),
            pl.BlockSpec(W.shape, lambda j, a: (0, 0)),
            pl.BlockSpec((hidden, 1), lambda j, a: (0, 0)),
        ],
        out_specs=pl.BlockSpec((1, hidden, lane_blk), lambda j, a: (a, 0, j)),
        out_shape=jax.ShapeDtypeStruct((na, hidden, bs), jnp.float32),
        compiler_params=pltpu.CompilerParams(
            dimension_semantics=("parallel", "parallel"),
        ),
    )(t_t, x_t, emb_table, W, b.reshape(hidden, 1))

    return out.transpose(2, 0, 1)


# speedup vs baseline: 26.0689x; 1.0395x over previous
import jax
import jax.numpy as jnp
from jax.experimental import pallas as pl
from jax.experimental.pallas import tpu as pltpu


def _fused_block(t_ref, x_ref, emb_ref, w_ref, b_ref, o_ref):
    ni = t_ref.shape[0]
    nf = x_ref.shape[0] // ni
    lanes = x_ref.shape[1]
    n_type, n_emb = emb_ref.shape
    hidden = w_ref.shape[1]
    a_idx = pl.program_id(1)

    emb_w = jnp.dot(
        emb_ref[...], w_ref[0:n_emb, :], preferred_element_type=jnp.float32
    )
    wc = jnp.concatenate([w_ref[n_emb:, :], emb_w], axis=0)
    wc_t = wc.T

    iota_t = jax.lax.broadcasted_iota(jnp.int32, (n_type, 1), 0)

    lt = 256
    for c in range(lanes // lt):
        sl = slice(c * lt, (c + 1) * lt)
        ta = t_ref[:, pl.ds(a_idx, 1), sl]
        acc = None
        for i in range(ni):
            xi = x_ref[i * nf:(i + 1) * nf, sl]
            ti = ta[i]
            oh = (ti == iota_t).astype(jnp.float32)
            rhs = jnp.concatenate([xi, oh], axis=0)
            hi = jnp.dot(wc_t, rhs, preferred_element_type=jnp.float32)
            acc = hi if acc is None else jnp.maximum(acc, hi)
        out = jnp.maximum(acc + b_ref[...], 0.0)
        o_ref[0, :, sl] = out


def kernel(item_type, item, emb_table, W, b):
    bs, na, ni = item_type.shape
    nf = item.shape[-1]
    hidden = W.shape[-1]

    x_t = jnp.transpose(item, (2, 3, 1, 0)).reshape(ni * nf, na * bs)
    t_t = jnp.transpose(item_type.astype(jnp.int32), (2, 1, 0))

    lane_blk = 4096
    jgrid = bs // lane_blk

    out = pl.pallas_call(
        _fused_block,
        grid=(jgrid, na),
        in_specs=[
            pl.BlockSpec((ni, na, lane_blk), lambda j, a: (0, 0, j)),
            pl.BlockSpec(
                (ni * nf, lane_blk), lambda j, a, _g=jgrid: (0, a * _g + j)
            ),
            pl.BlockSpec(emb_table.shape, lambda j, a: (0, 0)),
            pl.BlockSpec(W.shape, lambda j, a: (0, 0)),
            pl.BlockSpec((hidden, 1), lambda j, a: (0, 0)),
        ],
        out_specs=pl.BlockSpec((1, hidden, lane_blk), lambda j, a: (a, 0, j)),
        out_shape=jax.ShapeDtypeStruct((na, hidden, bs), jnp.float32),
        compiler_params=pltpu.CompilerParams(
            dimension_semantics=("parallel", "parallel"),
        ),
    )(t_t, x_t, emb_table, W, b.reshape(hidden, 1))

    return out.transpose(2, 0, 1)
